# Initial kernel scaffold; baseline (speedup 1.0000x reference)
import jax
import jax.numpy as jnp
from jax.experimental import pallas as pl


def kernel(x, adj, W_in, b_in, W_cheb, b_cheb, W_out, b_out):
    raise NotImplementedError("write your pallas kernel here")


# speedup vs baseline: 1.0251x; 1.0251x over previous
import jax
import jax.numpy as jnp
from jax.experimental import pallas as pl

_BC = 256

_INTERPRET = False


def _prep_kernel(adj_ref, abf_ref, deg_ref):
    a = adj_ref[...]
    abf_ref[...] = a.astype(jnp.bfloat16)
    s = jnp.sum(a, axis=0, keepdims=True)
    deg_ref[...] = jnp.broadcast_to(s, (8, s.shape[1]))


def _head_kernel(x_ref, w_ref, b_ref, degT_ref, x0_ref, x0p_ref, d_ref):
    d = jax.lax.rsqrt(jnp.maximum(degT_ref[...], 1.0))
    h = jax.lax.dot_general(x_ref[...], w_ref[...],
                            (((1,), (0,)), ((), ())),
                            preferred_element_type=jnp.float32)
    h = jnp.maximum(h + b_ref[...], 0.0)
    x0_ref[...] = h
    x0p_ref[...] = (h * d).astype(jnp.bfloat16)
    d_ref[...] = d


def _spmm_kernel(abf_ref, fp_ref, out_ref):
    out_ref[...] = jax.lax.dot_general(abf_ref[...], fp_ref[...],
                                       (((0,), (0,)), ((), ())),
                                       preferred_element_type=jnp.float32)


def _mid_kernel(agg_ref, d_ref, x1_ref, x1p_ref):
    d = d_ref[...]
    x1 = -(agg_ref[...] * d)
    x1_ref[...] = x1
    x1p_ref[...] = (x1 * d).astype(jnp.bfloat16)


def _tail_kernel(agg_ref, x1_ref, x0_ref, d_ref, wc_ref, bc_ref, wo_ref,
                 bo_ref, out_ref):
    d = d_ref[...]
    x0 = x0_ref[...]
    x1 = x1_ref[...]
    x2 = -2.0 * (agg_ref[...] * d) + x1 - x0
    wc = wc_ref[...]
    dg = lambda a, b: jax.lax.dot_general(
        a, b, (((1,), (0,)), ((), ())), preferred_element_type=jnp.float32)
    hid = wc.shape[1]
    hc = dg(x0, wc[0:hid]) + dg(x1, wc[hid:2 * hid]) + dg(x2, wc[2 * hid:3 * hid])
    hc = jnp.maximum(hc + bc_ref[...], 0.0)
    out_ref[...] = dg(hc, wo_ref[...]) + bo_ref[...]


def kernel(x, adj, W_in, b_in, W_cheb, b_cheb, W_out, b_out):
    n = adj.shape[0]
    dim = x.shape[1]
    nb = (n + _BC - 1) // _BC

    abf, deg8 = pl.pallas_call(
        _prep_kernel,
        grid=(nb,),
        in_specs=[pl.BlockSpec((n, _BC), lambda j: (0, j))],
        out_specs=[pl.BlockSpec((n, _BC), lambda j: (0, j)),
                   pl.BlockSpec((8, _BC), lambda j: (0, j))],
        out_shape=[jax.ShapeDtypeStruct((n, n), jnp.bfloat16),
                   jax.ShapeDtypeStruct((8, n), jnp.float32)],
        interpret=_INTERPRET,
    )(adj)

    degT = deg8[0:1].T

    x0, x0p, dcol = pl.pallas_call(
        _head_kernel,
        out_shape=[jax.ShapeDtypeStruct((n, dim), jnp.float32),
                   jax.ShapeDtypeStruct((n, dim), jnp.bfloat16),
                   jax.ShapeDtypeStruct((n, 1), jnp.float32)],
        interpret=_INTERPRET,
    )(x, W_in, b_in.reshape(1, -1), degT)

    def spmm(fp):
        return pl.pallas_call(
            _spmm_kernel,
            grid=(nb,),
            in_specs=[pl.BlockSpec((n, _BC), lambda j: (0, j)),
                      pl.BlockSpec((n, dim), lambda j: (0, 0))],
            out_specs=pl.BlockSpec((_BC, dim), lambda j: (j, 0)),
            out_shape=jax.ShapeDtypeStruct((n, dim), jnp.float32),
            interpret=_INTERPRET,
        )(abf, fp)

    agg1 = spmm(x0p)

    x1, x1p = pl.pallas_call(
        _mid_kernel,
        out_shape=[jax.ShapeDtypeStruct((n, dim), jnp.float32),
                   jax.ShapeDtypeStruct((n, dim), jnp.bfloat16)],
        interpret=_INTERPRET,
    )(agg1, dcol)

    agg2 = spmm(x1p)

    out = pl.pallas_call(
        _tail_kernel,
        out_shape=jax.ShapeDtypeStruct((n, dim), jnp.float32),
        interpret=_INTERPRET,
    )(agg2, x1, x0, dcol, W_cheb, b_cheb.reshape(1, -1), W_out,
      b_out.reshape(1, -1))

    return out


# speedup vs baseline: 1.2176x; 1.1878x over previous
import jax
import jax.numpy as jnp
from jax.experimental import pallas as pl

_BC = 256

_INTERPRET = False


def _prep_kernel(adj_ref, a8_ref, deg_ref):
    a = adj_ref[...]
    a8_ref[...] = a.astype(jnp.int8)
    s = jnp.sum(a, axis=0, keepdims=True)
    deg_ref[...] = jnp.broadcast_to(s, (8, s.shape[1]))


def _head_kernel(x_ref, w_ref, b_ref, degT_ref, x0_ref, x0p_ref, d_ref):
    d = jax.lax.rsqrt(jnp.maximum(degT_ref[...], 1.0))
    h = jax.lax.dot_general(x_ref[...], w_ref[...],
                            (((1,), (0,)), ((), ())),
                            preferred_element_type=jnp.float32)
    h = jnp.maximum(h + b_ref[...], 0.0)
    x0_ref[...] = h
    x0p_ref[...] = (h * d).astype(jnp.bfloat16)
    d_ref[...] = d


def _spmm_kernel(a8_ref, fp_ref, out_ref):
    ab = a8_ref[...].astype(jnp.bfloat16)
    out_ref[...] = jax.lax.dot_general(ab, fp_ref[...],
                                       (((0,), (0,)), ((), ())),
                                       preferred_element_type=jnp.float32)


def _mid_kernel(agg_ref, d_ref, x1_ref, x1p_ref):
    d = d_ref[...]
    x1 = -(agg_ref[...] * d)
    x1_ref[...] = x1
    x1p_ref[...] = (x1 * d).astype(jnp.bfloat16)


def _tail_kernel(agg_ref, x1_ref, x0_ref, d_ref, wc_ref, bc_ref, wo_ref,
                 bo_ref, out_ref):
    d = d_ref[...]
    x0 = x0_ref[...]
    x1 = x1_ref[...]
    x2 = -2.0 * (agg_ref[...] * d) + x1 - x0
    wc = wc_ref[...]
    dg = lambda a, b: jax.lax.dot_general(
        a, b, (((1,), (0,)), ((), ())), preferred_element_type=jnp.float32)
    hid = wc.shape[1]
    hc = dg(x0, wc[0:hid]) + dg(x1, wc[hid:2 * hid]) + dg(x2, wc[2 * hid:3 * hid])
    hc = jnp.maximum(hc + bc_ref[...], 0.0)
    out_ref[...] = dg(hc, wo_ref[...]) + bo_ref[...]


def kernel(x, adj, W_in, b_in, W_cheb, b_cheb, W_out, b_out):
    n = adj.shape[0]
    dim = x.shape[1]
    nb = (n + _BC - 1) // _BC

    a8, deg8 = pl.pallas_call(
        _prep_kernel,
        grid=(nb,),
        in_specs=[pl.BlockSpec((n, _BC), lambda j: (0, j))],
        out_specs=[pl.BlockSpec((n, _BC), lambda j: (0, j)),
                   pl.BlockSpec((8, _BC), lambda j: (0, j))],
        out_shape=[jax.ShapeDtypeStruct((n, n), jnp.int8),
                   jax.ShapeDtypeStruct((8, n), jnp.float32)],
        interpret=_INTERPRET,
    )(adj)

    degT = deg8[0:1].T

    f_like = jax.ShapeDtypeStruct((n, dim), jnp.float32)
    bf_like = jax.ShapeDtypeStruct((n, dim), jnp.bfloat16)

    x0, x0p, dcol = pl.pallas_call(
        _head_kernel,
        out_shape=[f_like, bf_like, jax.ShapeDtypeStruct((n, 1), jnp.float32)],
        interpret=_INTERPRET,
    )(x, W_in, b_in.reshape(1, -1), degT)

    def spmm(fp):
        return pl.pallas_call(
            _spmm_kernel,
            grid=(nb,),
            in_specs=[pl.BlockSpec((n, _BC), lambda j: (0, j)),
                      pl.BlockSpec((n, dim), lambda j: (0, 0))],
            out_specs=pl.BlockSpec((_BC, dim), lambda j: (j, 0)),
            out_shape=f_like,
            interpret=_INTERPRET,
        )(a8, fp)

    agg1 = spmm(x0p)

    x1, x1p = pl.pallas_call(
        _mid_kernel,
        out_shape=[f_like, bf_like],
        interpret=_INTERPRET,
    )(agg1, dcol)

    agg2 = spmm(x1p)

    out = pl.pallas_call(
        _tail_kernel,
        out_shape=f_like,
        interpret=_INTERPRET,
    )(agg2, x1, x0, dcol, W_cheb, b_cheb.reshape(1, -1), W_out,
      b_out.reshape(1, -1))

    return out


# speedup vs baseline: 1.2470x; 1.0241x over previous
import jax
import jax.numpy as jnp
from jax.experimental import pallas as pl
from jax.experimental.pallas import tpu as pltpu

_BP = 256
_BC = 512


def _bf16_dot(a8, fp):
    return jax.lax.dot_general(a8.astype(jnp.bfloat16), fp,
                               (((0,), (0,)), ((), ())),
                               preferred_element_type=jnp.float32)


def _prep_kernel(adj_ref, a8_ref, deg_ref):
    a = adj_ref[...]
    a8_ref[...] = a.astype(jnp.int8)
    s = jnp.sum(a, axis=0, keepdims=True)
    deg_ref[...] = jnp.broadcast_to(s, (8, s.shape[1]))


def _head_kernel(x_ref, w_ref, b_ref, degT_ref, x0_ref, x0p_ref, d_ref):
    d = jax.lax.rsqrt(jnp.maximum(degT_ref[...], 1.0))
    h = jax.lax.dot_general(x_ref[...], w_ref[...],
                            (((1,), (0,)), ((), ())),
                            preferred_element_type=jnp.float32)
    h = jnp.maximum(h + b_ref[...], 0.0)
    x0_ref[...] = h
    x0p_ref[...] = (h * d).astype(jnp.bfloat16)
    d_ref[...] = d


def _spmm1_kernel(a8_ref, fp_ref, d_ref, agg_ref, x1_ref, x1p_ref):
    j = pl.program_id(0)
    nb = pl.num_programs(0)
    agg_ref[pl.ds(j * _BC, _BC), :] = _bf16_dot(a8_ref[...], fp_ref[...])

    @pl.when(j == nb - 1)
    def _epilogue():
        d = d_ref[...]
        x1 = -(agg_ref[...] * d)
        x1_ref[...] = x1
        x1p_ref[...] = (x1 * d).astype(jnp.bfloat16)


def _spmm2_kernel(a8_ref, fp_ref, x1_ref, x0_ref, d_ref, wc_ref, bc_ref,
                  wo_ref, bo_ref, out_ref, acc_ref):
    j = pl.program_id(0)
    nb = pl.num_programs(0)
    acc_ref[pl.ds(j * _BC, _BC), :] = _bf16_dot(a8_ref[...], fp_ref[...])

    @pl.when(j == nb - 1)
    def _epilogue():
        d = d_ref[...]
        x0 = x0_ref[...]
        x1 = x1_ref[...]
        x2 = -2.0 * (acc_ref[...] * d) + x1 - x0
        wc = wc_ref[...]
        dg = lambda a, b: jax.lax.dot_general(
            a, b, (((1,), (0,)), ((), ())), preferred_element_type=jnp.float32)
        hid = wc.shape[1]
        hc = dg(x0, wc[0:hid]) + dg(x1, wc[hid:2 * hid]) \
            + dg(x2, wc[2 * hid:3 * hid])
        hc = jnp.maximum(hc + bc_ref[...], 0.0)
        out_ref[...] = dg(hc, wo_ref[...]) + bo_ref[...]


def kernel(x, adj, W_in, b_in, W_cheb, b_cheb, W_out, b_out):
    n = adj.shape[0]
    dim = x.shape[1]
    nbp = (n + _BP - 1) // _BP
    nb = (n + _BC - 1) // _BC
    np_ = nb * _BC

    a8, deg8 = pl.pallas_call(
        _prep_kernel,
        grid=(nbp,),
        in_specs=[pl.BlockSpec((n, _BP), lambda j: (0, j))],
        out_specs=[pl.BlockSpec((n, _BP), lambda j: (0, j)),
                   pl.BlockSpec((8, _BP), lambda j: (0, j))],
        out_shape=[jax.ShapeDtypeStruct((n, n), jnp.int8),
                   jax.ShapeDtypeStruct((8, n), jnp.float32)],
    )(adj)

    degT = jnp.pad(deg8[0:1], ((0, 0), (0, np_ - n)), constant_values=1.0).T
    x_pad = jnp.pad(x, ((0, np_ - n), (0, 0)))

    f_like = jax.ShapeDtypeStruct((np_, dim), jnp.float32)
    bf_like = jax.ShapeDtypeStruct((np_, dim), jnp.bfloat16)
    full = lambda r, c: pl.BlockSpec((r, c), lambda j: (0, 0))

    x0, x0p, dcol = pl.pallas_call(
        _head_kernel,
        out_shape=[f_like, bf_like,
                   jax.ShapeDtypeStruct((np_, 1), jnp.float32)],
    )(x_pad, W_in, b_in.reshape(1, -1), degT)

    agg1, x1, x1p = pl.pallas_call(
        _spmm1_kernel,
        grid=(nb,),
        in_specs=[pl.BlockSpec((n, _BC), lambda j: (0, j)),
                  full(n, dim), full(np_, 1)],
        out_specs=[full(np_, dim), full(np_, dim), full(np_, dim)],
        out_shape=[f_like, f_like, bf_like],
    )(a8, x0p[:n], dcol)

    out = pl.pallas_call(
        _spmm2_kernel,
        grid=(nb,),
        in_specs=[pl.BlockSpec((n, _BC), lambda j: (0, j)),
                  full(n, dim), full(np_, dim), full(np_, dim), full(np_, 1),
                  full(3 * dim, dim), full(1, dim), full(dim, dim),
                  full(1, dim)],
        out_specs=full(np_, dim),
        out_shape=f_like,
        scratch_shapes=[pltpu.VMEM((np_, dim), jnp.float32)],
    )(a8, x1p[:n], x1, x0, dcol, W_cheb, b_cheb.reshape(1, -1), W_out,
      b_out.reshape(1, -1))

    return out[:n]


# speedup vs baseline: 1.4687x; 1.1778x over previous
import jax
import jax.numpy as jnp
from jax.experimental import pallas as pl
from jax.experimental.pallas import tpu as pltpu

_BP = 256
_BC = 512


def _prep_kernel(adj_ref, a8_ref, deg_ref):
    a = adj_ref[...]
    pad = a8_ref.shape[0] - a.shape[0]
    a8_ref[...] = jnp.concatenate(
        [a.astype(jnp.int8), jnp.zeros((pad, a.shape[1]), jnp.int8)], axis=0)
    s = jnp.sum(a, axis=0, keepdims=True)
    deg_ref[...] = jnp.broadcast_to(s, (8, s.shape[1]))


def _head_kernel(x_ref, w_ref, b_ref, deg_ref, x0t_ref, x0pt_ref, dt_ref):
    d = jax.lax.rsqrt(jnp.maximum(deg_ref[...], 1.0))
    h = jax.lax.dot_general(x_ref[...], w_ref[...],
                            (((1,), (0,)), ((), ())),
                            preferred_element_type=jnp.float32)
    h = jnp.maximum(h + b_ref[...], 0.0)
    ht = jnp.transpose(h)
    x0t_ref[...] = ht
    x0pt_ref[...] = (ht * d).astype(jnp.bfloat16)
    dt_ref[...] = d


def _mxu(fpt, a8):
    return jax.lax.dot_general(fpt, a8.astype(jnp.bfloat16),
                               (((1,), (0,)), ((), ())),
                               preferred_element_type=jnp.float32)


def _spmm1_kernel(a8_ref, fpt_ref, dt_ref, aggt_ref, x1t_ref, x1pt_ref):
    j = pl.program_id(0)
    nb = pl.num_programs(0)
    aggt_ref[:, pl.ds(j * _BC, _BC)] = _mxu(fpt_ref[...], a8_ref[...])

    @pl.when(j == nb - 1)
    def _epilogue():
        d = dt_ref[...]
        x1t = -(aggt_ref[...] * d)
        x1t_ref[...] = x1t
        x1pt_ref[...] = (x1t * d).astype(jnp.bfloat16)


def _spmm2_kernel(a8_ref, fpt_ref, x1t_ref, x0t_ref, dt_ref, wc_ref, bc_ref,
                  wo_ref, bo_ref, out_ref, acc_ref):
    j = pl.program_id(0)
    nb = pl.num_programs(0)
    acc_ref[:, pl.ds(j * _BC, _BC)] = _mxu(fpt_ref[...], a8_ref[...])

    @pl.when(j == nb - 1)
    def _epilogue():
        d = dt_ref[...]
        x0t = x0t_ref[...]
        x1t = x1t_ref[...]
        x2t = -2.0 * (acc_ref[...] * d) + x1t - x0t
        wc = wc_ref[...]
        dgt = lambda w, xt: jax.lax.dot_general(
            w, xt, (((0,), (0,)), ((), ())), preferred_element_type=jnp.float32)
        hid = wc.shape[1]
        hct = dgt(wc[0:hid], x0t) + dgt(wc[hid:2 * hid], x1t) \
            + dgt(wc[2 * hid:3 * hid], x2t)
        hct = jnp.maximum(hct + bc_ref[...], 0.0)
        out_t = dgt(wo_ref[...], hct) + bo_ref[...]
        out_ref[...] = jnp.transpose(out_t)


def kernel(x, adj, W_in, b_in, W_cheb, b_cheb, W_out, b_out):
    n = adj.shape[0]
    dim = x.shape[1]
    nbp = (n + _BP - 1) // _BP
    nb = (n + _BC - 1) // _BC
    np_ = nb * _BC

    a8, deg8 = pl.pallas_call(
        _prep_kernel,
        grid=(nbp,),
        in_specs=[pl.BlockSpec((n, _BP), lambda j: (0, j))],
        out_specs=[pl.BlockSpec((np_, _BP), lambda j: (0, j)),
                   pl.BlockSpec((8, _BP), lambda j: (0, j))],
        out_shape=[jax.ShapeDtypeStruct((np_, n), jnp.int8),
                   jax.ShapeDtypeStruct((8, n), jnp.float32)],
    )(adj)

    deg_row = jnp.pad(deg8[0:1], ((0, 0), (0, np_ - n)), constant_values=1.0)
    x_pad = jnp.pad(x, ((0, np_ - n), (0, 0)))

    ft_like = jax.ShapeDtypeStruct((dim, np_), jnp.float32)
    bft_like = jax.ShapeDtypeStruct((dim, np_), jnp.bfloat16)
    full = lambda r, c: pl.BlockSpec((r, c), lambda j: (0, 0))

    x0t, x0pt, dt = pl.pallas_call(
        _head_kernel,
        out_shape=[ft_like, bft_like,
                   jax.ShapeDtypeStruct((1, np_), jnp.float32)],
    )(x_pad, W_in, b_in.reshape(1, -1), deg_row)

    aggt1, x1t, x1pt = pl.pallas_call(
        _spmm1_kernel,
        grid=(nb,),
        in_specs=[pl.BlockSpec((np_, _BC), lambda j: (0, j)),
                  full(dim, np_), full(1, np_)],
        out_specs=[full(dim, np_), full(dim, np_), full(dim, np_)],
        out_shape=[ft_like, ft_like, bft_like],
    )(a8, x0pt, dt)

    out = pl.pallas_call(
        _spmm2_kernel,
        grid=(nb,),
        in_specs=[pl.BlockSpec((np_, _BC), lambda j: (0, j)),
                  full(dim, np_), full(dim, np_), full(dim, np_),
                  full(1, np_), full(3 * dim, dim), full(dim, 1),
                  full(dim, dim), full(dim, 1)],
        out_specs=full(np_, dim),
        out_shape=jax.ShapeDtypeStruct((np_, dim), jnp.float32),
        scratch_shapes=[pltpu.VMEM((dim, np_), jnp.float32)],
    )(a8, x1pt, x1t, x0t, dt, W_cheb, b_cheb.reshape(-1, 1), W_out,
      b_out.reshape(-1, 1))

    return out[:n]


# speedup vs baseline: 1.5163x; 1.0325x over previous
import jax
import jax.numpy as jnp
from jax.experimental import pallas as pl
from jax.experimental.pallas import tpu as pltpu

_BP = 256
_BC = 512


def _prep_kernel(adj_ref, a8_ref, deg_ref):
    a = adj_ref[...]
    pad = a8_ref.shape[0] - a.shape[0]
    a8_ref[...] = jnp.concatenate(
        [a.astype(jnp.int8), jnp.zeros((pad, a.shape[1]), jnp.int8)], axis=0)
    s = jnp.sum(a, axis=0, keepdims=True)
    deg_ref[...] = jnp.broadcast_to(s, (8, s.shape[1]))


def _mxu(fpt, a8):
    return jax.lax.dot_general(fpt, a8.astype(jnp.bfloat16),
                               (((1,), (0,)), ((), ())),
                               preferred_element_type=jnp.float32)


def _spmm1_kernel(a8_ref, x_ref, w_ref, b_ref, deg_ref,
                  aggt_ref, x0t_ref, dt_ref, x1t_ref, x1pt_ref, fpt_ref):
    j = pl.program_id(0)
    nb = pl.num_programs(0)

    @pl.when(j == 0)
    def _head():
        d = jax.lax.rsqrt(jnp.maximum(deg_ref[...], 1.0))
        h = jax.lax.dot_general(x_ref[...], w_ref[...],
                                (((1,), (0,)), ((), ())),
                                preferred_element_type=jnp.float32)
        h = jnp.maximum(h + b_ref[...], 0.0)
        ht = jnp.transpose(h)
        x0t_ref[...] = ht
        fpt_ref[...] = (ht * d).astype(jnp.bfloat16)
        dt_ref[...] = d

    aggt_ref[:, pl.ds(j * _BC, _BC)] = _mxu(fpt_ref[...], a8_ref[...])

    @pl.when(j == nb - 1)
    def _epilogue():
        d = dt_ref[...]
        x1t = -(aggt_ref[...] * d)
        x1t_ref[...] = x1t
        x1pt_ref[...] = (x1t * d).astype(jnp.bfloat16)


def _spmm2_kernel(a8_ref, fpt_ref, x1t_ref, x0t_ref, dt_ref, wc_ref, bc_ref,
                  wo_ref, bo_ref, out_ref, acc_ref):
    j = pl.program_id(0)
    nb = pl.num_programs(0)
    acc_ref[:, pl.ds(j * _BC, _BC)] = _mxu(fpt_ref[...], a8_ref[...])

    @pl.when(j == nb - 1)
    def _epilogue():
        d = dt_ref[...]
        x0t = x0t_ref[...]
        x1t = x1t_ref[...]
        x2t = -2.0 * (acc_ref[...] * d) + x1t - x0t
        wc = wc_ref[...]
        dgt = lambda w, xt: jax.lax.dot_general(
            w, xt, (((0,), (0,)), ((), ())), preferred_element_type=jnp.float32)
        hid = wc.shape[1]
        hct = dgt(wc[0:hid], x0t) + dgt(wc[hid:2 * hid], x1t) \
            + dgt(wc[2 * hid:3 * hid], x2t)
        hct = jnp.maximum(hct + bc_ref[...], 0.0)
        out_t = dgt(wo_ref[...], hct) + bo_ref[...]
        n_out = out_ref.shape[0]
        out_ref[...] = jnp.transpose(out_t)[:n_out]


def kernel(x, adj, W_in, b_in, W_cheb, b_cheb, W_out, b_out):
    n = adj.shape[0]
    dim = x.shape[1]
    nbp = (n + _BP - 1) // _BP
    nb = (n + _BC - 1) // _BC
    np_ = nb * _BC

    a8, deg8 = pl.pallas_call(
        _prep_kernel,
        grid=(nbp,),
        in_specs=[pl.BlockSpec((n, _BP), lambda j: (0, j))],
        out_specs=[pl.BlockSpec((np_, _BP), lambda j: (0, j)),
                   pl.BlockSpec((8, _BP), lambda j: (0, j))],
        out_shape=[jax.ShapeDtypeStruct((np_, n), jnp.int8),
                   jax.ShapeDtypeStruct((8, n), jnp.float32)],
    )(adj)

    deg_row = jnp.pad(deg8[0:1], ((0, 0), (0, np_ - n)), constant_values=1.0)
    x_pad = jnp.pad(x, ((0, np_ - n), (0, 0)))

    ft_like = jax.ShapeDtypeStruct((dim, np_), jnp.float32)
    bft_like = jax.ShapeDtypeStruct((dim, np_), jnp.bfloat16)
    full = lambda r, c: pl.BlockSpec((r, c), lambda j: (0, 0))

    aggt1, x0t, dt, x1t, x1pt = pl.pallas_call(
        _spmm1_kernel,
        grid=(nb,),
        in_specs=[pl.BlockSpec((np_, _BC), lambda j: (0, j)),
                  full(np_, dim), full(dim, dim), full(1, dim), full(1, np_)],
        out_specs=[full(dim, np_), full(dim, np_), full(1, np_),
                   full(dim, np_), full(dim, np_)],
        out_shape=[ft_like, ft_like, jax.ShapeDtypeStruct((1, np_), jnp.float32),
                   ft_like, bft_like],
        scratch_shapes=[pltpu.VMEM((dim, np_), jnp.bfloat16)],
    )(a8, x_pad, W_in, b_in.reshape(1, -1), deg_row)

    out = pl.pallas_call(
        _spmm2_kernel,
        grid=(nb,),
        in_specs=[pl.BlockSpec((np_, _BC), lambda j: (0, j)),
                  full(dim, np_), full(dim, np_), full(dim, np_),
                  full(1, np_), full(3 * dim, dim), full(dim, 1),
                  full(dim, dim), full(dim, 1)],
        out_specs=full(n, dim),
        out_shape=jax.ShapeDtypeStruct((n, dim), jnp.float32),
        scratch_shapes=[pltpu.VMEM((dim, np_), jnp.float32)],
    )(a8, x1pt, x1t, x0t, dt, W_cheb, b_cheb.reshape(-1, 1), W_out,
      b_out.reshape(-1, 1))

    return out


# speedup vs baseline: 1.5189x; 1.0017x over previous
import jax
import jax.numpy as jnp
from jax.experimental import pallas as pl
from jax.experimental.pallas import tpu as pltpu

_BP = 512
_BC = 512


def _prep_kernel(adj_ref, a8_ref, deg_ref):
    a = adj_ref[...]
    pad = a8_ref.shape[0] - a.shape[0]
    a8_ref[...] = jnp.concatenate(
        [a.astype(jnp.int8), jnp.zeros((pad, a.shape[1]), jnp.int8)], axis=0)
    s = jnp.sum(a, axis=0, keepdims=True)
    deg_ref[...] = jnp.broadcast_to(s, (8, s.shape[1]))


def _mxu(fpt, a8):
    return jax.lax.dot_general(fpt, a8.astype(jnp.bfloat16),
                               (((1,), (0,)), ((), ())),
                               preferred_element_type=jnp.float32)


def _spmm1_kernel(a8_ref, x_ref, w_ref, b_ref, deg_ref,
                  aggt_ref, x0t_ref, dt_ref, x1t_ref, x1pt_ref, fpt_ref):
    j = pl.program_id(0)
    nb = pl.num_programs(0)

    @pl.when(j == 0)
    def _head():
        d = jax.lax.rsqrt(jnp.maximum(deg_ref[...], 1.0))
        h = jax.lax.dot_general(x_ref[...], w_ref[...],
                                (((1,), (0,)), ((), ())),
                                preferred_element_type=jnp.float32)
        h = jnp.maximum(h + b_ref[...], 0.0)
        ht = jnp.transpose(h)
        x0t_ref[...] = ht
        fpt_ref[...] = (ht * d).astype(jnp.bfloat16)
        dt_ref[...] = d

    aggt_ref[:, pl.ds(j * _BC, _BC)] = _mxu(fpt_ref[...], a8_ref[...])

    @pl.when(j == nb - 1)
    def _epilogue():
        d = dt_ref[...]
        x1t = -(aggt_ref[...] * d)
        x1t_ref[...] = x1t
        x1pt_ref[...] = (x1t * d).astype(jnp.bfloat16)


def _spmm2_kernel(a8_ref, fpt_ref, x1t_ref, x0t_ref, dt_ref, wc_ref, bc_ref,
                  wo_ref, bo_ref, out_ref, acc_ref):
    j = pl.program_id(0)
    nb = pl.num_programs(0)
    acc_ref[:, pl.ds(j * _BC, _BC)] = _mxu(fpt_ref[...], a8_ref[...])

    @pl.when(j == nb - 1)
    def _epilogue():
        d = dt_ref[...]
        x0t = x0t_ref[...]
        x1t = x1t_ref[...]
        x2t = -2.0 * (acc_ref[...] * d) + x1t - x0t
        wc = wc_ref[...]
        dgt = lambda w, xt: jax.lax.dot_general(
            w, xt, (((0,), (0,)), ((), ())), preferred_element_type=jnp.float32)
        hid = wc.shape[1]
        hct = dgt(wc[0:hid], x0t) + dgt(wc[hid:2 * hid], x1t) \
            + dgt(wc[2 * hid:3 * hid], x2t)
        hct = jnp.maximum(hct + bc_ref[...], 0.0)
        out_t = dgt(wo_ref[...], hct) + bo_ref[...]
        n_out = out_ref.shape[0]
        out_ref[...] = jnp.transpose(out_t)[:n_out]


def kernel(x, adj, W_in, b_in, W_cheb, b_cheb, W_out, b_out):
    n = adj.shape[0]
    dim = x.shape[1]
    nbp = (n + _BP - 1) // _BP
    nb = (n + _BC - 1) // _BC
    np_ = nb * _BC

    a8, deg8 = pl.pallas_call(
        _prep_kernel,
        grid=(nbp,),
        in_specs=[pl.BlockSpec((n, _BP), lambda j: (0, j))],
        out_specs=[pl.BlockSpec((np_, _BP), lambda j: (0, j)),
                   pl.BlockSpec((8, _BP), lambda j: (0, j))],
        out_shape=[jax.ShapeDtypeStruct((np_, n), jnp.int8),
                   jax.ShapeDtypeStruct((8, n), jnp.float32)],
    )(adj)

    deg_row = jnp.pad(deg8[0:1], ((0, 0), (0, np_ - n)), constant_values=1.0)
    x_pad = jnp.pad(x, ((0, np_ - n), (0, 0)))

    ft_like = jax.ShapeDtypeStruct((dim, np_), jnp.float32)
    bft_like = jax.ShapeDtypeStruct((dim, np_), jnp.bfloat16)
    full = lambda r, c: pl.BlockSpec((r, c), lambda j: (0, 0))

    aggt1, x0t, dt, x1t, x1pt = pl.pallas_call(
        _spmm1_kernel,
        grid=(nb,),
        in_specs=[pl.BlockSpec((np_, _BC), lambda j: (0, j)),
                  full(np_, dim), full(dim, dim), full(1, dim), full(1, np_)],
        out_specs=[full(dim, np_), full(dim, np_), full(1, np_),
                   full(dim, np_), full(dim, np_)],
        out_shape=[ft_like, ft_like, jax.ShapeDtypeStruct((1, np_), jnp.float32),
                   ft_like, bft_like],
        scratch_shapes=[pltpu.VMEM((dim, np_), jnp.bfloat16)],
    )(a8, x_pad, W_in, b_in.reshape(1, -1), deg_row)

    out = pl.pallas_call(
        _spmm2_kernel,
        grid=(nb,),
        in_specs=[pl.BlockSpec((np_, _BC), lambda j: (0, j)),
                  full(dim, np_), full(dim, np_), full(dim, np_),
                  full(1, np_), full(3 * dim, dim), full(dim, 1),
                  full(dim, dim), full(dim, 1)],
        out_specs=full(n, dim),
        out_shape=jax.ShapeDtypeStruct((n, dim), jnp.float32),
        scratch_shapes=[pltpu.VMEM((dim, np_), jnp.float32)],
    )(a8, x1pt, x1t, x0t, dt, W_cheb, b_cheb.reshape(-1, 1), W_out,
      b_out.reshape(-1, 1))

    return out


# speedup vs baseline: 1.5757x; 1.0374x over previous
import jax
import jax.numpy as jnp
from jax.experimental import pallas as pl
from jax.experimental.pallas import tpu as pltpu

_BP = 512
_BC = 1024


def _prep_kernel(adj_ref, a8_ref, deg_ref):
    a = adj_ref[...]
    pad = a8_ref.shape[0] - a.shape[0]
    a8_ref[...] = jnp.concatenate(
        [a.astype(jnp.int8), jnp.zeros((pad, a.shape[1]), jnp.int8)], axis=0)
    s = jnp.sum(a, axis=0, keepdims=True)
    deg_ref[...] = jnp.broadcast_to(s, (8, s.shape[1]))


def _mxu(fpt, a8):
    return jax.lax.dot_general(fpt, a8.astype(jnp.bfloat16),
                               (((1,), (0,)), ((), ())),
                               preferred_element_type=jnp.float32)


def _spmm1_kernel(a8_ref, x_ref, w_ref, b_ref, deg_ref,
                  aggt_ref, x0t_ref, dt_ref, x1t_ref, x1pt_ref, fpt_ref):
    j = pl.program_id(0)
    nb = pl.num_programs(0)

    @pl.when(j == 0)
    def _head():
        d = jax.lax.rsqrt(jnp.maximum(deg_ref[...], 1.0))
        h = jax.lax.dot_general(x_ref[...], w_ref[...],
                                (((1,), (0,)), ((), ())),
                                preferred_element_type=jnp.float32)
        h = jnp.maximum(h + b_ref[...], 0.0)
        ht = jnp.transpose(h)
        x0t_ref[...] = ht
        fpt_ref[...] = (ht * d).astype(jnp.bfloat16)
        dt_ref[...] = d

    aggt_ref[:, pl.ds(j * _BC, _BC)] = _mxu(fpt_ref[...], a8_ref[...])

    @pl.when(j == nb - 1)
    def _epilogue():
        d = dt_ref[...]
        x1t = -(aggt_ref[...] * d)
        x1t_ref[...] = x1t
        x1pt_ref[...] = (x1t * d).astype(jnp.bfloat16)


def _spmm2_kernel(a8_ref, fpt_ref, x1t_ref, x0t_ref, dt_ref, wc_ref, bc_ref,
                  wo_ref, bo_ref, out_ref, acc_ref):
    j = pl.program_id(0)
    nb = pl.num_programs(0)
    acc_ref[:, pl.ds(j * _BC, _BC)] = _mxu(fpt_ref[...], a8_ref[...])

    @pl.when(j == nb - 1)
    def _epilogue():
        d = dt_ref[...]
        x0t = x0t_ref[...]
        x1t = x1t_ref[...]
        x2t = -2.0 * (acc_ref[...] * d) + x1t - x0t
        wc = wc_ref[...]
        dgt = lambda w, xt: jax.lax.dot_general(
            w, xt, (((0,), (0,)), ((), ())), preferred_element_type=jnp.float32)
        hid = wc.shape[1]
        hct = dgt(wc[0:hid], x0t) + dgt(wc[hid:2 * hid], x1t) \
            + dgt(wc[2 * hid:3 * hid], x2t)
        hct = jnp.maximum(hct + bc_ref[...], 0.0)
        out_t = dgt(wo_ref[...], hct) + bo_ref[...]
        n_out = out_ref.shape[0]
        out_ref[...] = jnp.transpose(out_t)[:n_out]


def kernel(x, adj, W_in, b_in, W_cheb, b_cheb, W_out, b_out):
    n = adj.shape[0]
    dim = x.shape[1]
    nbp = (n + _BP - 1) // _BP
    nb = (n + _BC - 1) // _BC
    np_ = nb * _BC

    a8, deg8 = pl.pallas_call(
        _prep_kernel,
        grid=(nbp,),
        in_specs=[pl.BlockSpec((n, _BP), lambda j: (0, j))],
        out_specs=[pl.BlockSpec((np_, _BP), lambda j: (0, j)),
                   pl.BlockSpec((8, _BP), lambda j: (0, j))],
        out_shape=[jax.ShapeDtypeStruct((np_, n), jnp.int8),
                   jax.ShapeDtypeStruct((8, n), jnp.float32)],
    )(adj)

    deg_row = jnp.pad(deg8[0:1], ((0, 0), (0, np_ - n)), constant_values=1.0)
    x_pad = jnp.pad(x, ((0, np_ - n), (0, 0)))

    ft_like = jax.ShapeDtypeStruct((dim, np_), jnp.float32)
    bft_like = jax.ShapeDtypeStruct((dim, np_), jnp.bfloat16)
    full = lambda r, c: pl.BlockSpec((r, c), lambda j: (0, 0))

    aggt1, x0t, dt, x1t, x1pt = pl.pallas_call(
        _spmm1_kernel,
        grid=(nb,),
        in_specs=[pl.BlockSpec((np_, _BC), lambda j: (0, j)),
                  full(np_, dim), full(dim, dim), full(1, dim), full(1, np_)],
        out_specs=[full(dim, np_), full(dim, np_), full(1, np_),
                   full(dim, np_), full(dim, np_)],
        out_shape=[ft_like, ft_like, jax.ShapeDtypeStruct((1, np_), jnp.float32),
                   ft_like, bft_like],
        scratch_shapes=[pltpu.VMEM((dim, np_), jnp.bfloat16)],
    )(a8, x_pad, W_in, b_in.reshape(1, -1), deg_row)

    out = pl.pallas_call(
        _spmm2_kernel,
        grid=(nb,),
        in_specs=[pl.BlockSpec((np_, _BC), lambda j: (0, j)),
                  full(dim, np_), full(dim, np_), full(dim, np_),
                  full(1, np_), full(3 * dim, dim), full(dim, 1),
                  full(dim, dim), full(dim, 1)],
        out_specs=full(n, dim),
        out_shape=jax.ShapeDtypeStruct((n, dim), jnp.float32),
        scratch_shapes=[pltpu.VMEM((dim, np_), jnp.float32)],
    )(a8, x1pt, x1t, x0t, dt, W_cheb, b_cheb.reshape(-1, 1), W_out,
      b_out.reshape(-1, 1))

    return out
